# baseline (speedup 1.0000x reference)
import functools

import jax
import jax.numpy as jnp
from jax import lax
from jax.experimental import pallas as pl
from jax.experimental.pallas import tpu as pltpu
from jax.experimental.pallas import tpu_sc as plsc

VOCAB = 1000000
DIM = 16
NEG = 20
B = 4096
N = B * (1 + NEG)
NW = 32
BPW = B // NW
RPW = N // NW
IDXROWS = 24
NBLK = RPW // 16
LANE = 128
ROWS2D = N // LANE
POS_ROWS = B // LANE

_mesh = plsc.VectorSubcoreMesh(core_axis_name="c", subcore_axis_name="s")


@functools.partial(
    pl.kernel,
    out_type=jax.ShapeDtypeStruct((N,), jnp.float32),
    mesh=_mesh,
    scratch_types=[
        pltpu.VMEM((IDXROWS, LANE), jnp.int32),
        pltpu.VMEM((1, LANE), jnp.int32),
        pltpu.VMEM((IDXROWS, LANE), jnp.int32),
        pltpu.VMEM((1, LANE), jnp.int32),
        pltpu.VMEM((16, 8, DIM), jnp.float32),
        pltpu.VMEM((16, 8, DIM), jnp.float32),
        pltpu.VMEM((DIM, LANE), jnp.float32),
        pltpu.VMEM((RPW,), jnp.float32),
        pltpu.SemaphoreType.DMA,
        pltpu.SemaphoreType.DMA,
    ],
    compiler_params=pltpu.CompilerParams(needs_layout_passes=False),
)
def _sc_scores(emb_u, emb_v, ug, vg, us, vs, out,
               ug_v, vg_v, usub_v, vsub_v, ubuf, vbuf, cent,
               scores_v, usem, vsem):
    wid = lax.axis_index("s") * 2 + lax.axis_index("c")

    pltpu.sync_copy(ug.at[wid], ug_v)
    pltpu.sync_copy(vg.at[wid], vg_v)
    pltpu.sync_copy(us.at[wid], usub_v)
    pltpu.sync_copy(vs.at[wid], vsub_v)

    iota16 = lax.iota(jnp.int32, 16)

    def v_phase(ph, carry):
        gvec = vg_v[0, pl.ds(ph * 16, 16)]
        for k in range(16):
            pltpu.async_copy(emb_v.at[gvec[k]], vbuf.at[k], vsem)
        for k in range(16):
            pltpu.make_async_copy(emb_v.at[gvec[k]], vbuf.at[k], vsem).wait()
        svec = vsub_v[0, pl.ds(ph * 16, 16)]
        for d in range(DIM):
            cols = jnp.full((16,), d, jnp.int32)
            cent[d, pl.ds(ph * 16, 16)] = plsc.load_gather(
                vbuf, [iota16, svec, cols])
        return carry

    lax.fori_loop(0, BPW // 16, v_phase, 0)

    def u_block(j, carry):
        row = j * 16
        gvec = ug_v[row // LANE, pl.ds(row % LANE, 16)]
        for k in range(16):
            pltpu.async_copy(emb_u.at[gvec[k]], ubuf.at[k], usem)
        for k in range(16):
            pltpu.make_async_copy(emb_u.at[gvec[k]], ubuf.at[k], usem).wait()
        svec = usub_v[row // LANE, pl.ds(row % LANE, 16)]
        rvec = row + iota16
        bvec = jnp.where(rvec < BPW, rvec, (rvec - BPW) // NEG)
        acc = jnp.zeros((16,), jnp.float32)
        for d in range(DIM):
            cols = jnp.full((16,), d, jnp.int32)
            uc = plsc.load_gather(ubuf, [iota16, svec, cols])
            vc = plsc.load_gather(cent, [cols, bvec])
            acc = acc + uc * vc
        scores_v[pl.ds(row, 16)] = acc
        return carry

    lax.fori_loop(0, NBLK, u_block, 0)

    pltpu.sync_copy(scores_v.at[pl.ds(0, BPW)], out.at[pl.ds(wid * BPW, BPW)])
    pltpu.sync_copy(scores_v.at[pl.ds(BPW, RPW - BPW)],
                    out.at[pl.ds(B + wid * (RPW - BPW), RPW - BPW)])


def _tc_reduce_body(s_ref, o_ref):
    s = s_ref[...]
    ridx = lax.broadcasted_iota(jnp.int32, (ROWS2D, LANE), 0)
    t = jnp.where(ridx < POS_ROWS, s, -s)
    ls = jnp.minimum(t, 0.0) - jnp.log1p(jnp.exp(-jnp.abs(t)))
    o_ref[0, 0] = -jnp.sum(ls) / B


_tc_reduce = pl.pallas_call(
    _tc_reduce_body,
    out_shape=jax.ShapeDtypeStruct((1, 1), jnp.float32),
    out_specs=pl.BlockSpec(memory_space=pltpu.SMEM),
)


def _pad_rows(x2d, rows):
    out = jnp.zeros((NW, rows * LANE), x2d.dtype)
    out = lax.dynamic_update_slice(out, x2d, (0, 0))
    return out.reshape(NW, rows, LANE)


def kernel(embedding_v, embedding_u, center_words, target_words, negative_words):
    c = center_words.reshape(-1).astype(jnp.int32)
    t = target_words.reshape(-1).astype(jnp.int32)
    n = negative_words.reshape(-1).astype(jnp.int32)
    uidx = jnp.concatenate(
        [t.reshape(NW, BPW), n.reshape(NW, RPW - BPW)], axis=1)
    ug = _pad_rows(uidx // 8, IDXROWS)
    us = _pad_rows(uidx % 8, IDXROWS)
    vidx = c.reshape(NW, BPW)
    vg = (vidx // 8).reshape(NW, 1, LANE)
    vs = (vidx % 8).reshape(NW, 1, LANE)
    emb_u3 = embedding_u.reshape(VOCAB // 8, 8, DIM)
    emb_v3 = embedding_v.reshape(VOCAB // 8, 8, DIM)
    scores = _sc_scores(emb_u3, emb_v3, ug, vg, us, vs)
    loss = _tc_reduce(scores.reshape(ROWS2D, LANE))
    return loss[0, 0]
